# Initial kernel scaffold; baseline (speedup 1.0000x reference)
import functools

import jax
import jax.numpy as jnp
from jax import lax
from jax.experimental import pallas as pl
from jax.experimental.pallas import tpu as pltpu
from jax.experimental.pallas import tpu_sc as plsc

N = 50000
E = 800000
W = 32
DEPTH = 4

NC = 2
NS = 16
NWORK = NC * NS
EPW = E // NWORK
C = 1000
NCH = EPW // C
RPT = N // NS
ZR = 800

_mesh = plsc.VectorSubcoreMesh(core_axis_name="c", subcore_axis_name="s")


def _zero_fill(zref, nrows):
    z = jnp.zeros((16,), jnp.float32)

    @plsc.parallel_loop(0, nrows, 1, unroll=8)
    def _(i):
        zref[i, pl.ds(0, 16)] = z
        zref[i, pl.ds(16, 16)] = z


def _zero_shared(zeros_v, accum_sh, row_base, nrows):
    full, rem = nrows // ZR, nrows % ZR
    for k in range(full):
        pltpu.sync_copy(zeros_v, accum_sh.at[pl.ds(row_base + k * ZR, ZR)])
    if rem:
        pltpu.sync_copy(zeros_v.at[pl.ds(0, rem)],
                        accum_sh.at[pl.ds(row_base + full * ZR, rem)])


@functools.partial(
    pl.kernel,
    out_type=jax.ShapeDtypeStruct((NC, N, W), jnp.float32),
    mesh=_mesh,
    scratch_types=[
        pltpu.VMEM((C,), jnp.int32),
        pltpu.VMEM((C,), jnp.int32),
        pltpu.VMEM((C, W), jnp.float32),
        pltpu.VMEM((C, W), jnp.float32),
        pltpu.VMEM((ZR, W), jnp.float32),
        pltpu.VMEM_SHARED((N, W), jnp.float32),
        pltpu.SemaphoreType.DMA,
    ],
)
def _sc_msg_sum(h_hbm, src_hbm, dst_hbm, w_hbm, out_hbm,
                src_v, dst_v, w_v, rows_v, zeros_v, accum_sh, sem):
    cid = lax.axis_index("c")
    sid = lax.axis_index("s")
    wid = cid * NS + sid

    _zero_fill(zeros_v, ZR)
    _zero_shared(zeros_v, accum_sh, sid * RPT, RPT)
    plsc.subcore_barrier()

    def chunk(k, _):
        base = wid * EPW + k * C
        pltpu.sync_copy(src_hbm.at[pl.ds(base, C)], src_v)
        pltpu.sync_copy(dst_hbm.at[pl.ds(base, C)], dst_v)
        pltpu.sync_copy(w_hbm.at[pl.ds(base, C)], w_v)
        pltpu.async_copy(h_hbm.at[src_v], rows_v, sem).wait()

        @plsc.parallel_loop(0, C, 1, unroll=8)
        def _(i):
            rows_v[i, pl.ds(0, 16)] = (rows_v[i, pl.ds(0, 16)]
                                       * w_v[i, pl.ds(0, 16)])
            rows_v[i, pl.ds(16, 16)] = (rows_v[i, pl.ds(16, 16)]
                                        * w_v[i, pl.ds(16, 16)])

        pltpu.sync_copy(rows_v, accum_sh.at[dst_v], add=True)
        return 0

    lax.fori_loop(0, NCH, chunk, 0)
    plsc.subcore_barrier()
    pltpu.sync_copy(accum_sh.at[pl.ds(sid * RPT, RPT)],
                    out_hbm.at[cid].at[pl.ds(sid * RPT, RPT)])


@functools.partial(
    pl.kernel,
    out_type=jax.ShapeDtypeStruct((NC, N, 16), jnp.float32),
    mesh=_mesh,
    scratch_types=[
        pltpu.VMEM((C,), jnp.int32),
        pltpu.VMEM((C, 16), jnp.float32),
        pltpu.VMEM((ZR, 16), jnp.float32),
        pltpu.VMEM_SHARED((N, 16), jnp.float32),
        pltpu.SemaphoreType.DMA,
    ],
)
def _sc_counts(dst_hbm, out_hbm, dst_v, ones_v, zeros_v, accum_sh, sem):
    cid = lax.axis_index("c")
    sid = lax.axis_index("s")
    wid = cid * NS + sid

    one = jnp.ones((16,), jnp.float32)

    @plsc.parallel_loop(0, C, 1, unroll=8)
    def _(i):
        ones_v[i, pl.ds(0, 16)] = one

    z = jnp.zeros((16,), jnp.float32)

    @plsc.parallel_loop(0, ZR, 1, unroll=8)
    def _(i):
        zeros_v[i, pl.ds(0, 16)] = z

    full, rem = RPT // ZR, RPT % ZR
    for k in range(full):
        pltpu.sync_copy(zeros_v, accum_sh.at[pl.ds(sid * RPT + k * ZR, ZR)])
    if rem:
        pltpu.sync_copy(zeros_v.at[pl.ds(0, rem)],
                        accum_sh.at[pl.ds(sid * RPT + full * ZR, rem)])
    plsc.subcore_barrier()

    def chunk(k, _):
        base = wid * EPW + k * C
        pltpu.sync_copy(dst_hbm.at[pl.ds(base, C)], dst_v)
        pltpu.sync_copy(ones_v, accum_sh.at[dst_v], add=True)
        return 0

    lax.fori_loop(0, NCH, chunk, 0)
    plsc.subcore_barrier()
    pltpu.sync_copy(accum_sh.at[pl.ds(sid * RPT, RPT)],
                    out_hbm.at[cid].at[pl.ds(sid * RPT, RPT)])


RB = 2000


def _fc1_body(x_ref, w_ref, b_ref, o_ref):
    o_ref[...] = x_ref[...] @ w_ref[...] + b_ref[...]


def _tc_fc1(x, w, b):
    return pl.pallas_call(
        _fc1_body,
        grid=(N // RB,),
        in_specs=[
            pl.BlockSpec((RB, 3), lambda i: (i, 0)),
            pl.BlockSpec((3, W), lambda i: (0, 0)),
            pl.BlockSpec((1, W), lambda i: (0, 0)),
        ],
        out_specs=pl.BlockSpec((RB, W), lambda i: (i, 0)),
        out_shape=jax.ShapeDtypeStruct((N, W), jnp.float32),
    )(x, w, b)


def _update_body(h_ref, s_ref, cnt_ref, root_ref, bias_ref, o_ref):
    ssum = s_ref[0] + s_ref[1]
    c = cnt_ref[0, :, 0:1] + cnt_ref[1, :, 0:1]
    inv = 1.0 / jnp.maximum(c, 1.0)
    o_ref[...] = jnp.maximum(
        h_ref[...] @ root_ref[...] + ssum * inv + bias_ref[...], 0.0)


def _tc_update(h, s2, cnt2, root, bias):
    return pl.pallas_call(
        _update_body,
        grid=(N // RB,),
        in_specs=[
            pl.BlockSpec((RB, W), lambda i: (i, 0)),
            pl.BlockSpec((NC, RB, W), lambda i: (0, i, 0)),
            pl.BlockSpec((NC, RB, 16), lambda i: (0, i, 0)),
            pl.BlockSpec((W, W), lambda i: (0, 0)),
            pl.BlockSpec((1, W), lambda i: (0, 0)),
        ],
        out_specs=pl.BlockSpec((RB, W), lambda i: (i, 0)),
        out_shape=jax.ShapeDtypeStruct((N, W), jnp.float32),
    )(h, s2, cnt2, root, bias)


def _fc2_body(h_ref, w_ref, b_ref, o_ref):
    o_ref[...] = h_ref[...] @ w_ref[...] + b_ref[...]


def _tc_fc2(h, w, b):
    return pl.pallas_call(
        _fc2_body,
        grid=(N // RB,),
        in_specs=[
            pl.BlockSpec((RB, W), lambda i: (i, 0)),
            pl.BlockSpec((W, 1), lambda i: (0, 0)),
            pl.BlockSpec((1, 1), lambda i: (0, 0)),
        ],
        out_specs=pl.BlockSpec((RB, 1), lambda i: (i, 0)),
        out_shape=jax.ShapeDtypeStruct((N, 1), jnp.float32),
    )(h, w, b)


EB = 4000


def _emlp_body(ea_ref, k11w, k11b, k12w, k12b, k21w, k21b, k22w, k22b,
               k31w, k31b, k32w, k32b, o0, o1, o2):
    ea = ea_ref[...]
    u0 = jnp.maximum(ea @ k11w[...] + k11b[...], 0.0)
    o0[...] = u0 @ k12w[...] + k12b[...]
    u1 = jnp.maximum(ea @ k21w[...] + k21b[...], 0.0)
    o1[...] = u1 @ k22w[...] + k22b[...]
    u2 = jnp.maximum(ea @ k31w[...] + k31b[...], 0.0)
    o2[...] = u2 @ k32w[...] + k32b[...]


def _tc_edge_mlp(ea, p):
    h8 = W // 4
    specs = [pl.BlockSpec((EB, 3), lambda i: (i, 0))]
    args = [ea]
    for i in range(3):
        for nm, shp in ((f'k{i}1_w', (3, h8)), (f'k{i}1_b', (1, h8)),
                        (f'k{i}2_w', (h8, W)), (f'k{i}2_b', (1, W))):
            specs.append(pl.BlockSpec(shp, lambda i: (0, 0)))
            a = p[nm]
            args.append(a.reshape(shp) if a.ndim == 1 else a)
    return pl.pallas_call(
        _emlp_body,
        grid=(E // EB,),
        in_specs=specs,
        out_specs=[pl.BlockSpec((EB, W), lambda i: (i, 0))] * 3,
        out_shape=[jax.ShapeDtypeStruct((E, W), jnp.float32)] * 3,
    )(*args)


def kernel(x, edge_index, edge_attr, params):
    p = params
    src = edge_index[0].astype(jnp.int32)
    dst = edge_index[1].astype(jnp.int32)

    ws = _tc_edge_mlp(edge_attr, p)
    cnt2 = _sc_counts(dst)

    h = _tc_fc1(x, p['fc1_w'], p['fc1_b'].reshape(1, W))
    for _ in range(DEPTH):
        for i in range(3):
            s2 = _sc_msg_sum(h, src, dst, ws[i])
            h = _tc_update(h, s2, cnt2, p[f'root{i}'],
                           p[f'bias{i}'].reshape(1, W))
    return _tc_fc2(h, p['fc2_w'].reshape(W, 1), p['fc2_b'].reshape(1, 1))


# speedup vs baseline: 7.7001x; 7.7001x over previous
import functools

import jax
import jax.numpy as jnp
from jax import lax
from jax.experimental import pallas as pl
from jax.experimental.pallas import tpu as pltpu
from jax.experimental.pallas import tpu_sc as plsc

N = 50000
E = 800000
W = 32
DEPTH = 4

NC = 2
NS = 16
NWORK = NC * NS
EPW = E // NWORK
C = 200
NCH = EPW // C
NP_ = 50048
RPT = NP_ // NS

N4 = N // 4
NP4 = NP_ // 4
E4 = E // 4

_mesh = plsc.VectorSubcoreMesh(core_axis_name="c", subcore_axis_name="s")


@functools.partial(
    pl.kernel,
    out_type=jax.ShapeDtypeStruct((NC, NP_, W), jnp.float32),
    mesh=_mesh,
    compiler_params=pltpu.CompilerParams(use_tc_tiling_on_sc=False),
    scratch_types=[
        pltpu.VMEM((4, C), jnp.int32),
        pltpu.VMEM((4, C), jnp.int32),
        pltpu.VMEM((2, C * W), jnp.float32),
        pltpu.VMEM((2, C, W), jnp.float32),
        pltpu.VMEM_SHARED((NP_, W), jnp.float32),
    ] + [pltpu.SemaphoreType.DMA] * 8,
)
def _sc_msg_sum(h_hbm, src_hbm, dst_hbm, w_hbm, zero_hbm, out_hbm,
                src4, dst4, wv2, rows2, accum_sh,
                f0, f1, f2, f3, g0, g1, s0, s1):
    cid = lax.axis_index("c")
    sid = lax.axis_index("s")
    wid = cid * NS + sid
    fsem = [f0, f1, f2, f3]
    gsem = [g0, g1]
    ssem = [s0, s1]

    pltpu.sync_copy(zero_hbm.at[pl.ds(sid * RPT, RPT)],
                    accum_sh.at[pl.ds(sid * RPT, RPT)])
    plsc.subcore_barrier()

    def start_F(k, c):
        base = wid * EPW + k * C
        pltpu.async_copy(src_hbm.at[pl.ds(base, C)], src4.at[c % 4], fsem[c % 4])
        pltpu.async_copy(dst_hbm.at[pl.ds(base, C)], dst4.at[c % 4], fsem[c % 4])
        pltpu.async_copy(w_hbm.at[pl.ds(base * W, C * W)], wv2.at[c % 2],
                         fsem[c % 4])

    def wait_F(c):
        pltpu.make_async_copy(src_hbm.at[pl.ds(0, C)], src4.at[c % 4],
                              fsem[c % 4]).wait()
        pltpu.make_async_copy(dst_hbm.at[pl.ds(0, C)], dst4.at[c % 4],
                              fsem[c % 4]).wait()
        pltpu.make_async_copy(w_hbm.at[pl.ds(0, C * W)], wv2.at[c % 2],
                              fsem[c % 4]).wait()

    def start_G(c):
        pltpu.async_copy(h_hbm.at[src4.at[c % 4]], rows2.at[c % 2],
                         gsem[c % 2])

    def wait_G(c):
        pltpu.make_async_copy(h_hbm.at[src4.at[c % 4]], rows2.at[c % 2],
                              gsem[c % 2]).wait()

    def mult(c):
        b = c % 2

        @plsc.parallel_loop(0, C, 1, unroll=8)
        def _(i):
            rows2[b, i, pl.ds(0, 16)] = (rows2[b, i, pl.ds(0, 16)]
                                         * wv2[b, pl.ds(i * W, 16)])
            rows2[b, i, pl.ds(16, 16)] = (rows2[b, i, pl.ds(16, 16)]
                                          * wv2[b, pl.ds(i * W + 16, 16)])

    def start_S(c):
        pltpu.async_copy(rows2.at[c % 2], accum_sh.at[dst4.at[c % 4]],
                         ssem[c % 2], add=True)

    def wait_S(c):
        pltpu.make_async_copy(rows2.at[c % 2], accum_sh.at[dst4.at[c % 4]],
                              ssem[c % 2]).wait()

    start_F(0, 0)
    start_F(1, 1)
    wait_F(0)
    start_G(0)
    for c in range(4):
        if c > 0:
            wait_S(c - 1)
        wait_F(c + 1)
        start_G(c + 1)
        wait_G(c)
        mult(c)
        start_S(c)
        start_F(c + 2, c + 2)

    def quad(j, _):
        for c in range(4):
            k = 4 * j + c
            wait_S(c - 1)
            wait_F(c + 1)
            start_G(c + 1)
            wait_G(c)
            mult(c)
            start_S(c)
            if c == 3:
                @pl.when(j < (NCH - 2) // 4)
                def _():
                    start_F(k + 2, c + 2)
            else:
                start_F(k + 2, c + 2)
        return 0

    lax.fori_loop(1, NCH // 4, quad, 0)

    wait_S(3)
    wait_G(0)
    mult(0)
    start_S(0)
    wait_S(0)

    plsc.subcore_barrier()
    pltpu.sync_copy(accum_sh.at[pl.ds(sid * RPT, RPT)],
                    out_hbm.at[cid].at[pl.ds(sid * RPT, RPT)])


RB4 = 512


def _mm_body(x_ref, w_ref, b_ref, o_ref):
    o_ref[...] = x_ref[...] @ w_ref[...] + b_ref[...]


def _tc_matmul(x, w, b, rows, kin, kout):
    return pl.pallas_call(
        _mm_body,
        grid=(pl.cdiv(rows, RB4),),
        in_specs=[
            pl.BlockSpec((RB4, kin), lambda i: (i, 0)),
            pl.BlockSpec((kin, kout), lambda i: (0, 0)),
            pl.BlockSpec((1, kout), lambda i: (0, 0)),
        ],
        out_specs=pl.BlockSpec((RB4, kout), lambda i: (i, 0)),
        out_shape=jax.ShapeDtypeStruct((rows, kout), jnp.float32),
    )(x, w, b)


def _combine_body(r_ref, s_ref, cnt_ref, o_ref):
    ssum = s_ref[0] + s_ref[1]
    inv = 1.0 / jnp.maximum(cnt_ref[0] + cnt_ref[1], 1.0)
    o_ref[...] = jnp.maximum(r_ref[...] + ssum * inv, 0.0)


def _tc_combine(r, s2p, cnt2p):
    return pl.pallas_call(
        _combine_body,
        grid=(pl.cdiv(N4, RB4),),
        in_specs=[
            pl.BlockSpec((RB4, 128), lambda i: (i, 0)),
            pl.BlockSpec((NC, RB4, 128), lambda i: (0, i, 0)),
            pl.BlockSpec((NC, RB4, 128), lambda i: (0, i, 0)),
        ],
        out_specs=pl.BlockSpec((RB4, 128), lambda i: (i, 0)),
        out_shape=jax.ShapeDtypeStruct((N4, 128), jnp.float32),
    )(r, s2p, cnt2p)


EB4 = 2000


def _emlp_body(ea_ref, k11w, k11b, k12w, k12b, k21w, k21b, k22w, k22b,
               k31w, k31b, k32w, k32b, o0, o1, o2):
    ea = ea_ref[...]
    u0 = jnp.maximum(ea @ k11w[...] + k11b[...], 0.0)
    o0[...] = u0 @ k12w[...] + k12b[...]
    u1 = jnp.maximum(ea @ k21w[...] + k21b[...], 0.0)
    o1[...] = u1 @ k22w[...] + k22b[...]
    u2 = jnp.maximum(ea @ k31w[...] + k31b[...], 0.0)
    o2[...] = u2 @ k32w[...] + k32b[...]


def _tc_edge_mlp(eap, p):
    eye4 = jnp.eye(4, dtype=jnp.float32)
    specs = [pl.BlockSpec((EB4, 12), lambda i: (i, 0))]
    args = [eap]
    for i in range(3):
        k1w = jnp.kron(eye4, p[f'k{i}1_w'])
        k1b = jnp.tile(p[f'k{i}1_b'], 4).reshape(1, 32)
        k2w = jnp.kron(eye4, p[f'k{i}2_w'])
        k2b = jnp.tile(p[f'k{i}2_b'], 4).reshape(1, 128)
        for a in (k1w, k1b, k2w, k2b):
            sh = a.shape
            specs.append(pl.BlockSpec(sh, lambda i: (0, 0)))
            args.append(a)
    return pl.pallas_call(
        _emlp_body,
        grid=(E4 // EB4,),
        in_specs=specs,
        out_specs=[pl.BlockSpec((EB4, 128), lambda i: (i, 0))] * 3,
        out_shape=[jax.ShapeDtypeStruct((E4, 128), jnp.float32)] * 3,
    )(*args)


def kernel(x, edge_index, edge_attr, params):
    p = params
    src = edge_index[0].astype(jnp.int32)
    dst = edge_index[1].astype(jnp.int32)
    eye4 = jnp.eye(4, dtype=jnp.float32)

    wps = _tc_edge_mlp(edge_attr.reshape(E4, 12), p)
    w1ds = [wp.reshape(E * W) for wp in wps]
    zero = jnp.zeros((NP_, W), jnp.float32)

    cnt2 = _sc_msg_sum(jnp.ones((N, W), jnp.float32), src, dst,
                       jnp.ones((E * W,), jnp.float32), zero)
    cnt2p = cnt2.reshape(NC, NP4, 128)

    fc1B = jnp.kron(eye4, p['fc1_w'])
    fc1b = jnp.tile(p['fc1_b'], 4).reshape(1, 128)
    hp = _tc_matmul(x.reshape(N4, 12), fc1B, fc1b, N4, 12, 128)

    rootBs = [jnp.kron(eye4, p[f'root{i}']) for i in range(3)]
    biasPs = [jnp.tile(p[f'bias{i}'], 4).reshape(1, 128) for i in range(3)]

    for _ in range(DEPTH):
        for i in range(3):
            r = _tc_matmul(hp, rootBs[i], biasPs[i], N4, 128, 128)
            s2 = _sc_msg_sum(hp.reshape(N, W), src, dst, w1ds[i], zero)
            hp = _tc_combine(r, s2.reshape(NC, NP4, 128), cnt2p)

    fc2B = jnp.kron(eye4, p['fc2_w'])
    fc2b = jnp.tile(p['fc2_b'].reshape(1, 1), (1, 4))
    outp = _tc_matmul(hp, fc2B, fc2b, N4, 128, 4)
    return outp.reshape(N, 1)


# speedup vs baseline: 13.8518x; 1.7989x over previous
import functools

import jax
import jax.numpy as jnp
from jax import lax
from jax.experimental import pallas as pl
from jax.experimental.pallas import tpu as pltpu
from jax.experimental.pallas import tpu_sc as plsc

N = 50000
E = 800000
W = 32
DEPTH = 4

NC = 2
NS = 16
NWORK = NC * NS
EPW = E // NWORK
C = 200
NCH = EPW // C
NP_ = 50048
RPT = NP_ // NS

EB = 6400
QB = EB // 4
N4 = N // 4
NP4 = NP_ // 4
E4 = E // 4

_mesh = plsc.VectorSubcoreMesh(core_axis_name="c", subcore_axis_name="s")


@functools.partial(
    pl.kernel,
    out_type=jax.ShapeDtypeStruct((NC, NP_, W), jnp.float32),
    mesh=_mesh,
    compiler_params=pltpu.CompilerParams(use_tc_tiling_on_sc=False),
    scratch_types=[
        pltpu.VMEM((4, C), jnp.int32),
        pltpu.VMEM((4, C), jnp.int32),
        pltpu.VMEM((2, C, W), jnp.float32),
        pltpu.VMEM((2, C, W), jnp.float32),
        pltpu.VMEM_SHARED((NP_, W), jnp.float32),
    ] + [pltpu.SemaphoreType.DMA] * 9,
)
def _sc_msg_sum(h_hbm, src_hbm, dst_hbm, w_hbm, zero_hbm, out_hbm,
                src4, dst4, wv2, rows2, accum_sh,
                f0, f1, f2, f3, g0, g1, s0, s1, z0):
    cid = lax.axis_index("c")
    sid = lax.axis_index("s")
    wid = cid * NS + sid
    fsem = [f0, f1, f2, f3]
    gsem = [g0, g1]
    ssem = [s0, s1]

    zcp = pltpu.async_copy(zero_hbm.at[pl.ds(sid * RPT, RPT)],
                           accum_sh.at[pl.ds(sid * RPT, RPT)], z0)

    def start_F(k, c):
        base = wid * EPW + k * C
        pltpu.async_copy(src_hbm.at[pl.ds(base, C)], src4.at[c % 4], fsem[c % 4])
        pltpu.async_copy(dst_hbm.at[pl.ds(base, C)], dst4.at[c % 4], fsem[c % 4])
        g = base // EB
        rem = base % EB
        row = g * QB + rem % QB
        lane = (rem // QB) * W
        pltpu.async_copy(w_hbm.at[pl.ds(row, C), pl.ds(lane, W)],
                         wv2.at[c % 2], fsem[c % 4])

    def wait_F(c):
        pltpu.make_async_copy(src_hbm.at[pl.ds(0, C)], src4.at[c % 4],
                              fsem[c % 4]).wait()
        pltpu.make_async_copy(dst_hbm.at[pl.ds(0, C)], dst4.at[c % 4],
                              fsem[c % 4]).wait()
        pltpu.make_async_copy(w_hbm.at[pl.ds(0, C), pl.ds(0, W)],
                              wv2.at[c % 2], fsem[c % 4]).wait()

    def start_G(c):
        pltpu.async_copy(h_hbm.at[src4.at[c % 4]], rows2.at[c % 2],
                         gsem[c % 2])

    def wait_G(c):
        pltpu.make_async_copy(h_hbm.at[src4.at[c % 4]], rows2.at[c % 2],
                              gsem[c % 2]).wait()

    def mult(c):
        b = c % 2

        @plsc.parallel_loop(0, C, 1, unroll=8)
        def _(i):
            rows2[b, i, pl.ds(0, 16)] = (rows2[b, i, pl.ds(0, 16)]
                                         * wv2[b, i, pl.ds(0, 16)])
            rows2[b, i, pl.ds(16, 16)] = (rows2[b, i, pl.ds(16, 16)]
                                          * wv2[b, i, pl.ds(16, 16)])

    def start_S(c):
        pltpu.async_copy(rows2.at[c % 2], accum_sh.at[dst4.at[c % 4]],
                         ssem[c % 2], add=True)

    def wait_S(c):
        pltpu.make_async_copy(rows2.at[c % 2], accum_sh.at[dst4.at[c % 4]],
                              ssem[c % 2]).wait()

    start_F(0, 0)
    start_F(1, 1)
    wait_F(0)
    start_G(0)
    zcp.wait()
    plsc.subcore_barrier()
    for c in range(4):
        if c > 0:
            wait_S(c - 1)
        wait_F(c + 1)
        start_G(c + 1)
        wait_G(c)
        mult(c)
        start_S(c)
        start_F(c + 2, c + 2)

    def quad(j, _):
        for c in range(4):
            k = 4 * j + c
            wait_S(c - 1)
            wait_F(c + 1)
            start_G(c + 1)
            wait_G(c)
            mult(c)
            start_S(c)
            if c == 3:
                @pl.when(j < (NCH - 2) // 4)
                def _():
                    start_F(k + 2, c + 2)
            else:
                start_F(k + 2, c + 2)
        return 0

    lax.fori_loop(1, NCH // 4, quad, 0)

    wait_S(3)
    wait_G(0)
    mult(0)
    start_S(0)
    wait_S(0)

    plsc.subcore_barrier()
    pltpu.sync_copy(accum_sh.at[pl.ds(sid * RPT, RPT)],
                    out_hbm.at[cid].at[pl.ds(sid * RPT, RPT)])


@functools.partial(
    pl.kernel,
    out_type=jax.ShapeDtypeStruct((NC, NP_, W), jnp.float32),
    mesh=_mesh,
    compiler_params=pltpu.CompilerParams(use_tc_tiling_on_sc=False),
    scratch_types=[
        pltpu.VMEM((4, C), jnp.int32),
        pltpu.VMEM((C, W), jnp.float32),
        pltpu.VMEM_SHARED((NP_, W), jnp.float32),
    ] + [pltpu.SemaphoreType.DMA] * 8,
)
def _sc_counts(dst_hbm, zero_hbm, out_hbm, dst4, ones_v, accum_sh,
               f0, f1, f2, f3, s0, s1, s2, s3):
    cid = lax.axis_index("c")
    sid = lax.axis_index("s")
    wid = cid * NS + sid
    fsem = [f0, f1, f2, f3]
    ssem = [s0, s1, s2, s3]

    one = jnp.ones((16,), jnp.float32)

    @plsc.parallel_loop(0, C, 1, unroll=8)
    def _(i):
        ones_v[i, pl.ds(0, 16)] = one
        ones_v[i, pl.ds(16, 16)] = one

    pltpu.sync_copy(zero_hbm.at[pl.ds(sid * RPT, RPT)],
                    accum_sh.at[pl.ds(sid * RPT, RPT)])
    plsc.subcore_barrier()

    def start_F(k, c):
        pltpu.async_copy(dst_hbm.at[pl.ds(wid * EPW + k * C, C)],
                         dst4.at[c % 4], fsem[c % 4])

    def wait_F(c):
        pltpu.make_async_copy(dst_hbm.at[pl.ds(0, C)], dst4.at[c % 4],
                              fsem[c % 4]).wait()

    def start_S(c):
        pltpu.async_copy(ones_v, accum_sh.at[dst4.at[c % 4]],
                         ssem[c % 4], add=True)

    def wait_S(c):
        pltpu.make_async_copy(ones_v, accum_sh.at[dst4.at[c % 4]],
                              ssem[c % 4]).wait()

    for c in range(4):
        start_F(c, c)
    for c in range(4):
        wait_F(c)
        start_S(c)

    def quad(j, _):
        for c in range(4):
            k = 4 * j + c
            wait_S(c)
            start_F(k, c)
            wait_F(c)
            start_S(c)
        return 0

    lax.fori_loop(1, NCH // 4, quad, 0)

    wait_S(0)
    start_F(NCH - 1, 0)
    wait_F(0)
    start_S(0)
    wait_S(0)
    wait_S(1)
    wait_S(2)
    wait_S(3)

    plsc.subcore_barrier()
    pltpu.sync_copy(accum_sh.at[pl.ds(sid * RPT, RPT)],
                    out_hbm.at[cid].at[pl.ds(sid * RPT, RPT)])


RB4 = 512


def _mm_body(x_ref, w_ref, b_ref, o_ref):
    o_ref[...] = x_ref[...] @ w_ref[...] + b_ref[...]


def _tc_matmul(x, w, b, rows, kin, kout):
    return pl.pallas_call(
        _mm_body,
        grid=(pl.cdiv(rows, RB4),),
        in_specs=[
            pl.BlockSpec((RB4, kin), lambda i: (i, 0)),
            pl.BlockSpec((kin, kout), lambda i: (0, 0)),
            pl.BlockSpec((1, kout), lambda i: (0, 0)),
        ],
        out_specs=pl.BlockSpec((RB4, kout), lambda i: (i, 0)),
        out_shape=jax.ShapeDtypeStruct((rows, kout), jnp.float32),
    )(x, w, b)


def _inv_body(cnt_ref, o_ref):
    o_ref[...] = 1.0 / jnp.maximum(cnt_ref[0] + cnt_ref[1], 1.0)


def _tc_inv(cnt2p):
    return pl.pallas_call(
        _inv_body,
        grid=(pl.cdiv(N4, RB4),),
        in_specs=[pl.BlockSpec((NC, RB4, 128), lambda i: (0, i, 0))],
        out_specs=pl.BlockSpec((RB4, 128), lambda i: (i, 0)),
        out_shape=jax.ShapeDtypeStruct((N4, 128), jnp.float32),
    )(cnt2p)


def _update_body(h_ref, w_ref, b_ref, s_ref, inv_ref, o_ref):
    ssum = s_ref[0] + s_ref[1]
    o_ref[...] = jnp.maximum(
        h_ref[...] @ w_ref[...] + b_ref[...] + ssum * inv_ref[...], 0.0)


def _tc_update(hp, rootB, biasP, s2p, invp):
    return pl.pallas_call(
        _update_body,
        grid=(pl.cdiv(N4, RB4),),
        in_specs=[
            pl.BlockSpec((RB4, 128), lambda i: (i, 0)),
            pl.BlockSpec((128, 128), lambda i: (0, 0)),
            pl.BlockSpec((1, 128), lambda i: (0, 0)),
            pl.BlockSpec((NC, RB4, 128), lambda i: (0, i, 0)),
            pl.BlockSpec((RB4, 128), lambda i: (i, 0)),
        ],
        out_specs=pl.BlockSpec((RB4, 128), lambda i: (i, 0)),
        out_shape=jax.ShapeDtypeStruct((N4, 128), jnp.float32),
    )(hp, rootB, biasP, s2p, invp)


def _emlp_body(ea_ref, k11w, k11b, k12w, k12b, k21w, k21b, k22w, k22b,
               k31w, k31b, k32w, k32b, o0, o1, o2):
    eaT = ea_ref[...]

    def mm(rhs):
        return jax.lax.dot_general(eaT, rhs, (((0,), (0,)), ((), ())),
                                   preferred_element_type=jnp.float32)

    def pack(y):
        return jnp.concatenate([y[0:QB], y[QB:2 * QB],
                                y[2 * QB:3 * QB], y[3 * QB:4 * QB]], axis=1)

    u0 = jnp.maximum(mm(k11w[...]) + k11b[...], 0.0)
    o0[...] = pack(u0 @ k12w[...] + k12b[...])
    u1 = jnp.maximum(mm(k21w[...]) + k21b[...], 0.0)
    o1[...] = pack(u1 @ k22w[...] + k22b[...])
    u2 = jnp.maximum(mm(k31w[...]) + k31b[...], 0.0)
    o2[...] = pack(u2 @ k32w[...] + k32b[...])


def _tc_edge_mlp(eaT, p):
    specs = [pl.BlockSpec((3, EB), lambda i: (0, i))]
    args = [eaT]
    for i in range(3):
        k1w = p[f'k{i}1_w']
        k1b = p[f'k{i}1_b'].reshape(1, 8)
        k2w = p[f'k{i}2_w']
        k2b = p[f'k{i}2_b'].reshape(1, 32)
        for a in (k1w, k1b, k2w, k2b):
            sh = a.shape
            specs.append(pl.BlockSpec(sh, lambda i: (0, 0)))
            args.append(a)
    return pl.pallas_call(
        _emlp_body,
        grid=(E // EB,),
        in_specs=specs,
        out_specs=[pl.BlockSpec((EB // 4, 128), lambda i: (i, 0))] * 3,
        out_shape=[jax.ShapeDtypeStruct((E4, 128), jnp.float32)] * 3,
    )(*args)


def kernel(x, edge_index, edge_attr, params):
    p = params
    src = edge_index[0].astype(jnp.int32)
    dst = edge_index[1].astype(jnp.int32)
    eye4 = jnp.eye(4, dtype=jnp.float32)

    wps = _tc_edge_mlp(edge_attr.T, p)
    zero = jnp.zeros((NP_, W), jnp.float32)

    cnt2 = _sc_counts(dst, zero)
    invp = _tc_inv(cnt2.reshape(NC, NP4, 128))

    fc1B = jnp.kron(eye4, p['fc1_w'])
    fc1b = jnp.tile(p['fc1_b'], 4).reshape(1, 128)
    hp = _tc_matmul(x.reshape(N4, 12), fc1B, fc1b, N4, 12, 128)

    rootBs = [jnp.kron(eye4, p[f'root{i}']) for i in range(3)]
    biasPs = [jnp.tile(p[f'bias{i}'], 4).reshape(1, 128) for i in range(3)]

    for _ in range(DEPTH):
        for i in range(3):
            s2 = _sc_msg_sum(hp.reshape(N, W), src, dst, wps[i], zero)
            hp = _tc_update(hp, rootBs[i], biasPs[i],
                            s2.reshape(NC, NP4, 128), invp)

    fc2B = jnp.kron(eye4, p['fc2_w'])
    fc2b = jnp.tile(p['fc2_b'].reshape(1, 1), (1, 4))
    outp = _tc_matmul(hp, fc2B, fc2b, N4, 128, 4)
    return outp.reshape(N, 1)
